# baseline (speedup 1.0000x reference)
import jax
import jax.numpy as jnp
from jax.experimental import pallas as pl
from jax.experimental.pallas import tpu as pltpu


def _sel_kernel(mask_ref, sel_ref, nsel_ref):
    c = mask_ref.shape[-1]
    nz = mask_ref[...] != 0.0
    nzi = nz.astype(jnp.int32)
    row = jax.lax.broadcasted_iota(jnp.int32, (c, c), 0)
    col = jax.lax.broadcasted_iota(jnp.int32, (c, c), 1)
    rank = jnp.sum((nz & (col < row)).astype(jnp.int32), axis=1)
    m = nz & (jnp.broadcast_to(rank[None, :], (c, c)) == row)
    sel = jnp.sum(jnp.where(m, col, 0), axis=1)
    sel_ref[...] = sel.reshape(1, c)
    nsel_ref[...] = jnp.sum(nzi, axis=-1, keepdims=True)


_U = 6


def _copy_kernel(sel_ref, nsel_ref, *refs):
    del sel_ref
    ins = refs[:_U]
    out_ref = refs[_U]
    k = pl.program_id(0)
    nsel = nsel_ref[0]
    for u in range(_U):
        j = _U * k + u

        @pl.when(j < nsel)
        def _valid(u=u):
            out_ref[:, u : u + 1] = ins[u][...]

        @pl.when(j >= nsel)
        def _invalid(u=u):
            out_ref[:, u : u + 1] = jnp.full_like(ins[u], jnp.nan)


def kernel(input_tensor, indexes):
    n, c, h, w = input_tensor.shape

    sel, nsel = pl.pallas_call(
        _sel_kernel,
        out_shape=(
            jax.ShapeDtypeStruct((1, c), jnp.int32),
            jax.ShapeDtypeStruct((1, 1), jnp.int32),
        ),
    )(indexes.reshape(1, c))
    sel = sel.reshape(c)
    nsel = nsel.reshape(1)

    def _in_spec(u):
        return pl.BlockSpec(
            (n, 1, h, w),
            lambda k, sel_ref, nsel_ref: (0, sel_ref[_U * k + u], 0, 0),
        )

    grid_spec = pltpu.PrefetchScalarGridSpec(
        num_scalar_prefetch=2,
        grid=(c // _U,),
        in_specs=[_in_spec(u) for u in range(_U)],
        out_specs=pl.BlockSpec(
            (n, _U, h, w), lambda k, sel_ref, nsel_ref: (0, k, 0, 0)
        ),
    )
    return pl.pallas_call(
        _copy_kernel,
        grid_spec=grid_spec,
        out_shape=jax.ShapeDtypeStruct((n, c, h, w), input_tensor.dtype),
        compiler_params=pltpu.CompilerParams(
            dimension_semantics=("parallel",),
        ),
    )(sel, nsel, *([input_tensor] * _U))


# speedup vs baseline: 1.0017x; 1.0017x over previous
import jax
import jax.numpy as jnp
from jax.experimental import pallas as pl
from jax.experimental.pallas import tpu as pltpu

_U = 8


def _sel_kernel(mask_ref, sel_ref, nsel_ref):
    c = mask_ref.shape[-1]
    nz = mask_ref[...] != 0.0
    nzi = nz.astype(jnp.int32)
    row = jax.lax.broadcasted_iota(jnp.int32, (c, c), 0)
    col = jax.lax.broadcasted_iota(jnp.int32, (c, c), 1)
    rank = jnp.sum((nz & (col < row)).astype(jnp.int32), axis=1)
    m = nz & (jnp.broadcast_to(rank[None, :], (c, c)) == row)
    sel = jnp.sum(jnp.where(m, col, 0), axis=1)
    sel_ref[...] = sel.reshape(1, c)
    nsel_ref[...] = jnp.sum(nzi, axis=-1, keepdims=True)


def _copy_kernel(sel_ref, nsel_ref, *refs):
    del sel_ref
    ins = refs[:_U]
    out_hbm = refs[_U]
    nan_buf = refs[_U + 1]
    wsem = refs[_U + 2]
    k = pl.program_id(0)
    nsel = nsel_ref[0]

    @pl.when(k == 0)
    def _fill_nan():
        nan_buf[...] = jnp.full_like(nan_buf, jnp.nan)

    for u in range(_U):
        j = k * _U + u
        dst = out_hbm.at[:, pl.ds(j, 1)]

        @pl.when(j < nsel)
        def _valid(u=u, dst=dst):
            pltpu.make_async_copy(ins[u], dst, wsem.at[u]).start()

        @pl.when(j >= nsel)
        def _invalid(dst=dst, u=u):
            pltpu.make_async_copy(nan_buf, dst, wsem.at[u]).start()

    for u in range(_U):
        j = k * _U + u
        pltpu.make_async_copy(
            ins[u], out_hbm.at[:, pl.ds(j, 1)], wsem.at[u]
        ).wait()


def kernel(input_tensor, indexes):
    n, c, h, w = input_tensor.shape

    sel, nsel = pl.pallas_call(
        _sel_kernel,
        out_shape=(
            jax.ShapeDtypeStruct((1, c), jnp.int32),
            jax.ShapeDtypeStruct((1, 1), jnp.int32),
        ),
    )(indexes.reshape(1, c))
    sel = sel.reshape(c)
    nsel = nsel.reshape(1)

    def _in_spec(u):
        return pl.BlockSpec(
            (n, 1, h, w),
            lambda k, sel_ref, nsel_ref: (0, sel_ref[_U * k + u], 0, 0),
        )

    grid_spec = pltpu.PrefetchScalarGridSpec(
        num_scalar_prefetch=2,
        grid=(c // _U,),
        in_specs=[_in_spec(u) for u in range(_U)],
        out_specs=pl.BlockSpec(memory_space=pltpu.MemorySpace.HBM),
        scratch_shapes=[
            pltpu.VMEM((n, 1, h, w), input_tensor.dtype),
            pltpu.SemaphoreType.DMA((_U,)),
        ],
    )
    return pl.pallas_call(
        _copy_kernel,
        grid_spec=grid_spec,
        out_shape=jax.ShapeDtypeStruct((n, c, h, w), input_tensor.dtype),
    )(sel, nsel, *([input_tensor] * _U))


# speedup vs baseline: 1.0025x; 1.0008x over previous
import jax
import jax.numpy as jnp
from jax.experimental import pallas as pl
from jax.experimental.pallas import tpu as pltpu

_U = 12


def _sel_kernel(mask_ref, sel_ref, nsel_ref):
    c = mask_ref.shape[-1]
    nz = mask_ref[...] != 0.0
    nzi = nz.astype(jnp.int32)
    row = jax.lax.broadcasted_iota(jnp.int32, (c, c), 0)
    col = jax.lax.broadcasted_iota(jnp.int32, (c, c), 1)
    rank = jnp.sum((nz & (col < row)).astype(jnp.int32), axis=1)
    m = nz & (jnp.broadcast_to(rank[None, :], (c, c)) == row)
    sel = jnp.sum(jnp.where(m, col, 0), axis=1)
    sel_ref[...] = sel.reshape(1, c)
    nsel_ref[...] = jnp.sum(nzi, axis=-1, keepdims=True)


def _copy_kernel(sel_ref, nsel_ref, *refs):
    del sel_ref
    ins = refs[:_U]
    out_hbm = refs[_U]
    nan_buf = refs[_U + 1]
    wsem = refs[_U + 2]
    k = pl.program_id(0)
    nsel = nsel_ref[0]

    @pl.when(k == 0)
    def _fill_nan():
        nan_buf[...] = jnp.full_like(nan_buf, jnp.nan)

    for u in range(_U):
        j = k * _U + u
        dst = out_hbm.at[:, pl.ds(j, 1)]

        @pl.when(j < nsel)
        def _valid(u=u, dst=dst):
            pltpu.make_async_copy(ins[u], dst, wsem.at[u]).start()

        @pl.when(j >= nsel)
        def _invalid(dst=dst, u=u):
            pltpu.make_async_copy(nan_buf, dst, wsem.at[u]).start()

    for u in range(_U):
        j = k * _U + u
        pltpu.make_async_copy(
            ins[u], out_hbm.at[:, pl.ds(j, 1)], wsem.at[u]
        ).wait()


def kernel(input_tensor, indexes):
    n, c, h, w = input_tensor.shape

    sel, nsel = pl.pallas_call(
        _sel_kernel,
        out_shape=(
            jax.ShapeDtypeStruct((1, c), jnp.int32),
            jax.ShapeDtypeStruct((1, 1), jnp.int32),
        ),
    )(indexes.reshape(1, c))
    sel = sel.reshape(c)
    nsel = nsel.reshape(1)

    def _in_spec(u):
        return pl.BlockSpec(
            (n, 1, h, w),
            lambda k, sel_ref, nsel_ref: (0, sel_ref[_U * k + u], 0, 0),
        )

    grid_spec = pltpu.PrefetchScalarGridSpec(
        num_scalar_prefetch=2,
        grid=(c // _U,),
        in_specs=[_in_spec(u) for u in range(_U)],
        out_specs=pl.BlockSpec(memory_space=pltpu.MemorySpace.HBM),
        scratch_shapes=[
            pltpu.VMEM((n, 1, h, w), input_tensor.dtype),
            pltpu.SemaphoreType.DMA((_U,)),
        ],
    )
    return pl.pallas_call(
        _copy_kernel,
        grid_spec=grid_spec,
        out_shape=jax.ShapeDtypeStruct((n, c, h, w), input_tensor.dtype),
    )(sel, nsel, *([input_tensor] * _U))
